# baseline (speedup 1.0000x reference)
import functools

import jax
import jax.numpy as jnp
from jax import lax
from jax.experimental import pallas as pl
from jax.experimental.pallas import tpu as pltpu
from jax.experimental.pallas import tpu_sc as plsc

N_NODES = 10000
N_EDGES = 320000
D_FEAT = 128

NC = 2
NS = 16
NW = NC * NS
L = 16

E_PER_W = N_EDGES // NW
CHUNK = 80
N_CHUNKS = E_PER_W // CHUNK
N_GROUPS = CHUNK // L
N_PAIRS = (N_CHUNKS - 1) // 2


def _sc_sqnorms_body(x_hbm, ei_hbm, sqn_hbm, si_v, di_v,
                     sr0, dr0, sr1, dr1, sqn_v,
                     sem_s0, sem_d0, sem_s1, sem_d1):
    wid = lax.axis_index("s") * NC + lax.axis_index("c")

    pltpu.sync_copy(ei_hbm.at[pl.ds(wid * E_PER_W, E_PER_W)], si_v)
    pltpu.sync_copy(
        ei_hbm.at[pl.ds(N_EDGES + wid * E_PER_W, E_PER_W)], di_v)

    def issue(it, sr, dr, sem_s, sem_d):
        pltpu.async_copy(x_hbm.at[si_v.at[pl.ds(it * CHUNK, CHUNK)]], sr, sem_s)
        pltpu.async_copy(x_hbm.at[di_v.at[pl.ds(it * CHUNK, CHUNK)]], dr, sem_d)

    def drain(it, sr, dr, sem_s, sem_d):
        pltpu.make_async_copy(
            x_hbm.at[si_v.at[pl.ds(it * CHUNK, CHUNK)]], sr, sem_s).wait()
        pltpu.make_async_copy(
            x_hbm.at[di_v.at[pl.ds(it * CHUNK, CHUNK)]], dr, sem_d).wait()

    def compute(it, sr, dr):
        @plsc.parallel_loop(0, N_GROUPS)
        def _group(g):
            rows = lax.iota(jnp.int32, L) + g * L
            diag = lax.iota(jnp.int32, L)
            z = jnp.zeros((L,), jnp.float32)
            accs = [z, z, z, z]
            for f in range(D_FEAT):
                col = (diag + f) & (D_FEAT - 1)
                s = plsc.load_gather(sr, [rows, col])
                d = plsc.load_gather(dr, [rows, col])
                t = s - d
                accs[f % 4] = accs[f % 4] + t * t
            sqn_v[pl.ds(it * CHUNK + g * L, L)] = (
                (accs[0] + accs[1]) + (accs[2] + accs[3]))

    issue(0, sr0, dr0, sem_s0, sem_d0)

    def pair_body(p, carry):
        a = 2 * p
        compute(a, sr0, dr0)
        compute(a + 1, sr1, dr1)
        return carry

    drain(0, sr0, dr0, sem_s0, sem_d0)
    lax.fori_loop(0, N_PAIRS, pair_body, 0)
    compute(N_CHUNKS - 1, sr0, dr0)

    pltpu.sync_copy(sqn_v, sqn_hbm.at[pl.ds(wid * E_PER_W, E_PER_W)])


_sc_sqnorms = functools.partial(
    pl.kernel,
    out_type=jax.ShapeDtypeStruct((N_EDGES,), jnp.float32),
    mesh=plsc.VectorSubcoreMesh(core_axis_name="c", subcore_axis_name="s",
                                num_cores=NC, num_subcores=NS),
    compiler_params=pltpu.CompilerParams(needs_layout_passes=False),
    scratch_types=[
        pltpu.VMEM((E_PER_W,), jnp.int32),
        pltpu.VMEM((E_PER_W,), jnp.int32),
        pltpu.VMEM((CHUNK, D_FEAT), jnp.float32),
        pltpu.VMEM((CHUNK, D_FEAT), jnp.float32),
        pltpu.VMEM((CHUNK, D_FEAT), jnp.float32),
        pltpu.VMEM((CHUNK, D_FEAT), jnp.float32),
        pltpu.VMEM((E_PER_W,), jnp.float32),
        pltpu.SemaphoreType.DMA,
        pltpu.SemaphoreType.DMA,
        pltpu.SemaphoreType.DMA,
        pltpu.SemaphoreType.DMA,
    ],
)(_sc_sqnorms_body)


def _mean_sqrt_body(sq_ref, out_ref):
    s = jnp.sum(jnp.sqrt(sq_ref[...])) * (1.0 / N_EDGES)
    out_ref[...] = jnp.full((1, 1), s, dtype=jnp.float32)


def kernel(x, edge_index):
    sqn = _sc_sqnorms(x, edge_index.reshape(2 * N_EDGES))
    out = pl.pallas_call(
        _mean_sqrt_body,
        out_shape=jax.ShapeDtypeStruct((1, 1), jnp.float32),
    )(sqn.reshape(N_EDGES // D_FEAT, D_FEAT))
    return out[0, 0]


# speedup vs baseline: 1.2789x; 1.2789x over previous
import functools

import jax
import jax.numpy as jnp
from jax import lax
from jax.experimental import pallas as pl
from jax.experimental.pallas import tpu as pltpu
from jax.experimental.pallas import tpu_sc as plsc

N_NODES = 10000
N_EDGES = 320000
D_FEAT = 128

NC = 2
NS = 16
NW = NC * NS
L = 16

E_PER_W = N_EDGES // NW
CHUNK = 80
N_CHUNKS = E_PER_W // CHUNK
N_GROUPS = CHUNK // L
N_PAIRS = (N_CHUNKS - 1) // 2
W = D_FEAT // 2


def _sc_sqnorms_body(x_hbm, ei_hbm, sqn_hbm, si_v, di_v,
                     sr0, dr0, sr1, dr1, sqn_v,
                     sem_s0, sem_d0, sem_s1, sem_d1):
    wid = lax.axis_index("s") * NC + lax.axis_index("c")

    pltpu.sync_copy(ei_hbm.at[pl.ds(wid * E_PER_W, E_PER_W)], si_v)
    pltpu.sync_copy(
        ei_hbm.at[pl.ds(N_EDGES + wid * E_PER_W, E_PER_W)], di_v)

    def issue(it, sr, dr, sem_s, sem_d):
        pltpu.async_copy(x_hbm.at[si_v.at[pl.ds(it * CHUNK, CHUNK)]], sr, sem_s)
        pltpu.async_copy(x_hbm.at[di_v.at[pl.ds(it * CHUNK, CHUNK)]], dr, sem_d)

    def drain(it, sr, dr, sem_s, sem_d):
        pltpu.make_async_copy(
            x_hbm.at[si_v.at[pl.ds(it * CHUNK, CHUNK)]], sr, sem_s).wait()
        pltpu.make_async_copy(
            x_hbm.at[di_v.at[pl.ds(it * CHUNK, CHUNK)]], dr, sem_d).wait()

    def compute(it, sr, dr):
        @plsc.parallel_loop(0, N_GROUPS)
        def _group(g):
            rows = lax.iota(jnp.int32, L) + g * L
            diag = lax.iota(jnp.int32, L)
            z = jnp.zeros((L,), jnp.float32)
            accs = [z, z, z, z]
            for wc in range(W):
                col = (diag + wc) & (W - 1)
                sw = plsc.load_gather(sr, [rows, col])
                dw = plsc.load_gather(dr, [rows, col])
                tb = plsc.bitcast(sw, jnp.bfloat16) - plsc.bitcast(
                    dw, jnp.bfloat16)
                u0, u1 = plsc.unpack(tb, format=plsc.PackFormat.INTERLEAVED)
                k = 2 * (wc % 2)
                accs[k] = accs[k] + u0 * u0
                accs[k + 1] = accs[k + 1] + u1 * u1
            sqn_v[pl.ds(it * CHUNK + g * L, L)] = (
                (accs[0] + accs[1]) + (accs[2] + accs[3]))

    issue(0, sr0, dr0, sem_s0, sem_d0)

    def pair_body(p, carry):
        a = 2 * p
        issue(a + 1, sr1, dr1, sem_s1, sem_d1)
        drain(a, sr0, dr0, sem_s0, sem_d0)
        compute(a, sr0, dr0)
        issue(a + 2, sr0, dr0, sem_s0, sem_d0)
        drain(a + 1, sr1, dr1, sem_s1, sem_d1)
        compute(a + 1, sr1, dr1)
        return carry

    lax.fori_loop(0, N_PAIRS, pair_body, 0)
    drain(N_CHUNKS - 1, sr0, dr0, sem_s0, sem_d0)
    compute(N_CHUNKS - 1, sr0, dr0)

    pltpu.sync_copy(sqn_v, sqn_hbm.at[pl.ds(wid * E_PER_W, E_PER_W)])


_sc_sqnorms = functools.partial(
    pl.kernel,
    out_type=jax.ShapeDtypeStruct((N_EDGES,), jnp.float32),
    mesh=plsc.VectorSubcoreMesh(core_axis_name="c", subcore_axis_name="s",
                                num_cores=NC, num_subcores=NS),
    compiler_params=pltpu.CompilerParams(needs_layout_passes=False,
                                         use_tc_tiling_on_sc=False),
    scratch_types=[
        pltpu.VMEM((E_PER_W,), jnp.int32),
        pltpu.VMEM((E_PER_W,), jnp.int32),
        pltpu.VMEM((CHUNK, W), jnp.int32),
        pltpu.VMEM((CHUNK, W), jnp.int32),
        pltpu.VMEM((CHUNK, W), jnp.int32),
        pltpu.VMEM((CHUNK, W), jnp.int32),
        pltpu.VMEM((E_PER_W,), jnp.float32),
        pltpu.SemaphoreType.DMA,
        pltpu.SemaphoreType.DMA,
        pltpu.SemaphoreType.DMA,
        pltpu.SemaphoreType.DMA,
    ],
)(_sc_sqnorms_body)


def _mean_sqrt_body(sq_ref, out_ref):
    s = jnp.sum(jnp.sqrt(sq_ref[...])) * (1.0 / N_EDGES)
    out_ref[...] = jnp.full((1, 1), s, dtype=jnp.float32)


def kernel(x, edge_index):
    xw = jax.lax.bitcast_convert_type(
        x.astype(jnp.bfloat16).reshape(N_NODES, W, 2), jnp.int32)
    sqn = _sc_sqnorms(xw, edge_index.reshape(2 * N_EDGES))
    out = pl.pallas_call(
        _mean_sqrt_body,
        out_shape=jax.ShapeDtypeStruct((1, 1), jnp.float32),
    )(sqn.reshape(N_EDGES // D_FEAT, D_FEAT))
    return out[0, 0]


# speedup vs baseline: 1.3062x; 1.0214x over previous
import functools

import jax
import jax.numpy as jnp
from jax import lax
from jax.experimental import pallas as pl
from jax.experimental.pallas import tpu as pltpu
from jax.experimental.pallas import tpu_sc as plsc

N_NODES = 10000
N_EDGES = 320000
D_FEAT = 128

NC = 2
NS = 16
NW = NC * NS
L = 16

E_PER_W = N_EDGES // NW
CHUNK = 80
N_CHUNKS = E_PER_W // CHUNK
N_GROUPS = CHUNK // L
N_PAIRS = (N_CHUNKS - 1) // 2
W = D_FEAT // 2


def _sc_sqnorms_body(x_hbm, ei_hbm, sqn_hbm, si_v, di_v,
                     sr0, dr0, sr1, dr1, sqn_v,
                     sem_s0, sem_d0, sem_s1, sem_d1):
    wid = lax.axis_index("s") * NC + lax.axis_index("c")

    pltpu.sync_copy(ei_hbm.at[pl.ds(wid * E_PER_W, E_PER_W)], si_v)
    pltpu.sync_copy(
        ei_hbm.at[pl.ds(N_EDGES + wid * E_PER_W, E_PER_W)], di_v)

    def issue(it, sr, dr, sem_s, sem_d):
        pltpu.async_copy(x_hbm.at[si_v.at[pl.ds(it * CHUNK, CHUNK)]], sr, sem_s)
        pltpu.async_copy(x_hbm.at[di_v.at[pl.ds(it * CHUNK, CHUNK)]], dr, sem_d)

    def drain(it, sr, dr, sem_s, sem_d):
        pltpu.make_async_copy(
            x_hbm.at[si_v.at[pl.ds(it * CHUNK, CHUNK)]], sr, sem_s).wait()
        pltpu.make_async_copy(
            x_hbm.at[di_v.at[pl.ds(it * CHUNK, CHUNK)]], dr, sem_d).wait()

    def compute(it, sr, dr):
        @plsc.parallel_loop(0, N_GROUPS)
        def _group(g):
            rows = lax.iota(jnp.int32, L) + g * L
            diag = lax.iota(jnp.int32, L)
            z = jnp.zeros((L,), jnp.float32)
            accs = [z, z, z, z]
            for wc in range(W):
                col = (diag + wc) & (W - 1)
                sw = plsc.load_gather(sr, [rows, col])
                dw = plsc.load_gather(dr, [rows, col])
                tb = plsc.bitcast(sw, jnp.bfloat16) - plsc.bitcast(
                    dw, jnp.bfloat16)
                tb2 = tb * tb
                u0, u1 = plsc.unpack(tb2, format=plsc.PackFormat.INTERLEAVED)
                k = 2 * (wc % 2)
                accs[k] = accs[k] + u0
                accs[k + 1] = accs[k + 1] + u1
            sqn_v[pl.ds(it * CHUNK + g * L, L)] = (
                (accs[0] + accs[1]) + (accs[2] + accs[3]))

    issue(0, sr0, dr0, sem_s0, sem_d0)

    def pair_body(p, carry):
        a = 2 * p
        issue(a + 1, sr1, dr1, sem_s1, sem_d1)
        drain(a, sr0, dr0, sem_s0, sem_d0)
        compute(a, sr0, dr0)
        issue(a + 2, sr0, dr0, sem_s0, sem_d0)
        drain(a + 1, sr1, dr1, sem_s1, sem_d1)
        compute(a + 1, sr1, dr1)
        return carry

    lax.fori_loop(0, N_PAIRS, pair_body, 0)
    drain(N_CHUNKS - 1, sr0, dr0, sem_s0, sem_d0)
    compute(N_CHUNKS - 1, sr0, dr0)

    pltpu.sync_copy(sqn_v, sqn_hbm.at[pl.ds(wid * E_PER_W, E_PER_W)])


_sc_sqnorms = functools.partial(
    pl.kernel,
    out_type=jax.ShapeDtypeStruct((N_EDGES,), jnp.float32),
    mesh=plsc.VectorSubcoreMesh(core_axis_name="c", subcore_axis_name="s",
                                num_cores=NC, num_subcores=NS),
    compiler_params=pltpu.CompilerParams(needs_layout_passes=False,
                                         use_tc_tiling_on_sc=False),
    scratch_types=[
        pltpu.VMEM((E_PER_W,), jnp.int32),
        pltpu.VMEM((E_PER_W,), jnp.int32),
        pltpu.VMEM((CHUNK, W), jnp.int32),
        pltpu.VMEM((CHUNK, W), jnp.int32),
        pltpu.VMEM((CHUNK, W), jnp.int32),
        pltpu.VMEM((CHUNK, W), jnp.int32),
        pltpu.VMEM((E_PER_W,), jnp.float32),
        pltpu.SemaphoreType.DMA,
        pltpu.SemaphoreType.DMA,
        pltpu.SemaphoreType.DMA,
        pltpu.SemaphoreType.DMA,
    ],
)(_sc_sqnorms_body)


def _mean_sqrt_body(sq_ref, out_ref):
    s = jnp.sum(jnp.sqrt(sq_ref[...])) * (1.0 / N_EDGES)
    out_ref[...] = jnp.full((1, 1), s, dtype=jnp.float32)


def kernel(x, edge_index):
    xw = jax.lax.bitcast_convert_type(
        x.astype(jnp.bfloat16).reshape(N_NODES, W, 2), jnp.int32)
    sqn = _sc_sqnorms(xw, edge_index.reshape(2 * N_EDGES))
    out = pl.pallas_call(
        _mean_sqrt_body,
        out_shape=jax.ShapeDtypeStruct((1, 1), jnp.float32),
    )(sqn.reshape(N_EDGES // D_FEAT, D_FEAT))
    return out[0, 0]


# speedup vs baseline: 1.4942x; 1.1439x over previous
import functools

import jax
import jax.numpy as jnp
from jax import lax
from jax.experimental import pallas as pl
from jax.experimental.pallas import tpu as pltpu
from jax.experimental.pallas import tpu_sc as plsc

N_NODES = 10000
N_EDGES = 320000
D_FEAT = 128

NC = 2
NS = 16
NW = NC * NS
L = 16

E_PER_W = N_EDGES // NW
CHUNK = 80
N_CHUNKS = E_PER_W // CHUNK
N_GROUPS = CHUNK // L
N_PAIRS = (N_CHUNKS - 1) // 2
W = D_FEAT // 2


def _sc_sqnorms_body(x_hbm, ei_hbm, sqn_hbm, si_v, di_v,
                     sr0, dr0, sr1, dr1, sr2, dr2, sqn_v,
                     sem_s0, sem_d0, sem_s1, sem_d1, sem_s2, sem_d2):
    wid = lax.axis_index("s") * NC + lax.axis_index("c")

    pltpu.sync_copy(ei_hbm.at[pl.ds(wid * E_PER_W, E_PER_W)], si_v)
    pltpu.sync_copy(
        ei_hbm.at[pl.ds(N_EDGES + wid * E_PER_W, E_PER_W)], di_v)

    def issue(it, sr, dr, sem_s, sem_d):
        pltpu.async_copy(x_hbm.at[si_v.at[pl.ds(it * CHUNK, CHUNK)]], sr, sem_s)
        pltpu.async_copy(x_hbm.at[di_v.at[pl.ds(it * CHUNK, CHUNK)]], dr, sem_d)

    def drain(it, sr, dr, sem_s, sem_d):
        pltpu.make_async_copy(
            x_hbm.at[si_v.at[pl.ds(it * CHUNK, CHUNK)]], sr, sem_s).wait()
        pltpu.make_async_copy(
            x_hbm.at[di_v.at[pl.ds(it * CHUNK, CHUNK)]], dr, sem_d).wait()

    def compute(it, sr, dr):
        @plsc.parallel_loop(0, N_GROUPS)
        def _group(g):
            rows = lax.iota(jnp.int32, L) + g * L
            diag = lax.iota(jnp.int32, L)
            z = jnp.zeros((L,), jnp.float32)
            accs = [z, z, z, z]
            for wc in range(W):
                col = (diag + wc) & (W - 1)
                sw = plsc.load_gather(sr, [rows, col])
                dw = plsc.load_gather(dr, [rows, col])
                tb = plsc.bitcast(sw, jnp.bfloat16) - plsc.bitcast(
                    dw, jnp.bfloat16)
                tb2 = tb * tb
                u0, u1 = plsc.unpack(tb2, format=plsc.PackFormat.INTERLEAVED)
                k = 2 * (wc % 2)
                accs[k] = accs[k] + u0
                accs[k + 1] = accs[k + 1] + u1
            sqn_v[pl.ds(it * CHUNK + g * L, L)] = (
                (accs[0] + accs[1]) + (accs[2] + accs[3]))

    bufs = ((sr0, dr0, sem_s0, sem_d0),
            (sr1, dr1, sem_s1, sem_d1),
            (sr2, dr2, sem_s2, sem_d2))
    issue(0, *bufs[0])
    issue(1, *bufs[1])

    def triple_body(p, carry):
        c = 3 * p
        for q in range(3):
            issue(c + q + 2, *bufs[(q + 2) % 3])
            drain(c + q, *bufs[q])
            compute(c + q, bufs[q][0], bufs[q][1])
        return carry

    lax.fori_loop(0, (N_CHUNKS - 2) // 3, triple_body, 0)
    drain(N_CHUNKS - 2, *bufs[0])
    compute(N_CHUNKS - 2, sr0, dr0)
    drain(N_CHUNKS - 1, *bufs[1])
    compute(N_CHUNKS - 1, sr1, dr1)

    pltpu.sync_copy(sqn_v, sqn_hbm.at[pl.ds(wid * E_PER_W, E_PER_W)])


_sc_sqnorms = functools.partial(
    pl.kernel,
    out_type=jax.ShapeDtypeStruct((N_EDGES,), jnp.float32),
    mesh=plsc.VectorSubcoreMesh(core_axis_name="c", subcore_axis_name="s",
                                num_cores=NC, num_subcores=NS),
    compiler_params=pltpu.CompilerParams(needs_layout_passes=False,
                                         use_tc_tiling_on_sc=False),
    scratch_types=[
        pltpu.VMEM((E_PER_W,), jnp.int32),
        pltpu.VMEM((E_PER_W,), jnp.int32),
        pltpu.VMEM((CHUNK, W), jnp.int32),
        pltpu.VMEM((CHUNK, W), jnp.int32),
        pltpu.VMEM((CHUNK, W), jnp.int32),
        pltpu.VMEM((CHUNK, W), jnp.int32),
        pltpu.VMEM((CHUNK, W), jnp.int32),
        pltpu.VMEM((CHUNK, W), jnp.int32),
        pltpu.VMEM((E_PER_W,), jnp.float32),
        pltpu.SemaphoreType.DMA,
        pltpu.SemaphoreType.DMA,
        pltpu.SemaphoreType.DMA,
        pltpu.SemaphoreType.DMA,
        pltpu.SemaphoreType.DMA,
        pltpu.SemaphoreType.DMA,
    ],
)(_sc_sqnorms_body)


def _mean_sqrt_body(sq_ref, out_ref):
    s = jnp.sum(jnp.sqrt(sq_ref[...])) * (1.0 / N_EDGES)
    out_ref[...] = jnp.full((1, 1), s, dtype=jnp.float32)


def kernel(x, edge_index):
    xw = jax.lax.bitcast_convert_type(
        x.astype(jnp.bfloat16).reshape(N_NODES, W, 2), jnp.int32)
    sqn = _sc_sqnorms(xw, edge_index.reshape(2 * N_EDGES))
    out = pl.pallas_call(
        _mean_sqrt_body,
        out_shape=jax.ShapeDtypeStruct((1, 1), jnp.float32),
    )(sqn.reshape(N_EDGES // D_FEAT, D_FEAT))
    return out[0, 0]


# speedup vs baseline: 1.5791x; 1.0568x over previous
import functools

import jax
import jax.numpy as jnp
from jax import lax
from jax.experimental import pallas as pl
from jax.experimental.pallas import tpu as pltpu
from jax.experimental.pallas import tpu_sc as plsc

N_NODES = 10000
N_EDGES = 320000
D_FEAT = 128

NC = 2
NS = 16
NW = NC * NS
L = 16

E_PER_W = N_EDGES // NW
CHUNK = 80
N_CHUNKS = E_PER_W // CHUNK
N_GROUPS = CHUNK // L
N_PAIRS = (N_CHUNKS - 1) // 2
W = D_FEAT // 2


def _sc_sqnorms_body(x_hbm, ei_hbm, sqn_hbm, si_v, di_v,
                     sr0, dr0, sr1, dr1, sr2, dr2, sqn_v,
                     sem_s0, sem_d0, sem_s1, sem_d1, sem_s2, sem_d2):
    wid = lax.axis_index("s") * NC + lax.axis_index("c")

    pltpu.sync_copy(ei_hbm.at[pl.ds(wid * E_PER_W, E_PER_W)], si_v)
    pltpu.sync_copy(
        ei_hbm.at[pl.ds(N_EDGES + wid * E_PER_W, E_PER_W)], di_v)

    def issue(it, sr, dr, sem_s, sem_d):
        pltpu.async_copy(x_hbm.at[si_v.at[pl.ds(it * CHUNK, CHUNK)]], sr, sem_s)
        pltpu.async_copy(x_hbm.at[di_v.at[pl.ds(it * CHUNK, CHUNK)]], dr, sem_d)

    def drain(it, sr, dr, sem_s, sem_d):
        pltpu.make_async_copy(
            x_hbm.at[si_v.at[pl.ds(it * CHUNK, CHUNK)]], sr, sem_s).wait()
        pltpu.make_async_copy(
            x_hbm.at[di_v.at[pl.ds(it * CHUNK, CHUNK)]], dr, sem_d).wait()

    def compute(it, sr, dr):
        @plsc.parallel_loop(0, N_GROUPS)
        def _group(g):
            rows = lax.iota(jnp.int32, L) + g * L
            diag = lax.iota(jnp.int32, L)
            z = jnp.zeros((L,), jnp.float32)

            @plsc.parallel_loop(0, W // 16, carry=(z, z, z, z))
            def blk_accs(blk, accs):
                a0, a1, a2, a3 = accs
                new = [a0, a1, a2, a3]
                base = blk * 16
                for j in range(16):
                    col = ((diag + j) & 15) + base
                    sw = plsc.load_gather(sr, [rows, col])
                    dw = plsc.load_gather(dr, [rows, col])
                    tb = plsc.bitcast(sw, jnp.bfloat16) - plsc.bitcast(
                        dw, jnp.bfloat16)
                    tb2 = tb * tb
                    u0, u1 = plsc.unpack(tb2,
                                         format=plsc.PackFormat.INTERLEAVED)
                    k = 2 * (j % 2)
                    new[k] = new[k] + u0
                    new[k + 1] = new[k + 1] + u1
                return tuple(new)

            a0, a1, a2, a3 = blk_accs
            sqn_v[pl.ds(it * CHUNK + g * L, L)] = (a0 + a1) + (a2 + a3)

    bufs = ((sr0, dr0, sem_s0, sem_d0),
            (sr1, dr1, sem_s1, sem_d1),
            (sr2, dr2, sem_s2, sem_d2))
    issue(0, *bufs[0])
    issue(1, *bufs[1])

    def triple_body(p, carry):
        c = 3 * p
        for q in range(3):
            issue(c + q + 2, *bufs[(q + 2) % 3])
            drain(c + q, *bufs[q])
            compute(c + q, bufs[q][0], bufs[q][1])
        return carry

    lax.fori_loop(0, (N_CHUNKS - 2) // 3, triple_body, 0)
    drain(N_CHUNKS - 2, *bufs[0])
    compute(N_CHUNKS - 2, sr0, dr0)
    drain(N_CHUNKS - 1, *bufs[1])
    compute(N_CHUNKS - 1, sr1, dr1)

    pltpu.sync_copy(sqn_v, sqn_hbm.at[pl.ds(wid * E_PER_W, E_PER_W)])


_sc_sqnorms = functools.partial(
    pl.kernel,
    out_type=jax.ShapeDtypeStruct((N_EDGES,), jnp.float32),
    mesh=plsc.VectorSubcoreMesh(core_axis_name="c", subcore_axis_name="s",
                                num_cores=NC, num_subcores=NS),
    compiler_params=pltpu.CompilerParams(needs_layout_passes=False,
                                         use_tc_tiling_on_sc=False),
    scratch_types=[
        pltpu.VMEM((E_PER_W,), jnp.int32),
        pltpu.VMEM((E_PER_W,), jnp.int32),
        pltpu.VMEM((CHUNK, W), jnp.int32),
        pltpu.VMEM((CHUNK, W), jnp.int32),
        pltpu.VMEM((CHUNK, W), jnp.int32),
        pltpu.VMEM((CHUNK, W), jnp.int32),
        pltpu.VMEM((CHUNK, W), jnp.int32),
        pltpu.VMEM((CHUNK, W), jnp.int32),
        pltpu.VMEM((E_PER_W,), jnp.float32),
        pltpu.SemaphoreType.DMA,
        pltpu.SemaphoreType.DMA,
        pltpu.SemaphoreType.DMA,
        pltpu.SemaphoreType.DMA,
        pltpu.SemaphoreType.DMA,
        pltpu.SemaphoreType.DMA,
    ],
)(_sc_sqnorms_body)


def _mean_sqrt_body(sq_ref, out_ref):
    s = jnp.sum(jnp.sqrt(sq_ref[...])) * (1.0 / N_EDGES)
    out_ref[...] = jnp.full((1, 1), s, dtype=jnp.float32)


def kernel(x, edge_index):
    xw = jax.lax.bitcast_convert_type(
        x.astype(jnp.bfloat16).reshape(N_NODES, W, 2), jnp.int32)
    sqn = _sc_sqnorms(xw, edge_index.reshape(2 * N_EDGES))
    out = pl.pallas_call(
        _mean_sqrt_body,
        out_shape=jax.ShapeDtypeStruct((1, 1), jnp.float32),
    )(sqn.reshape(N_EDGES // D_FEAT, D_FEAT))
    return out[0, 0]
